# baseline (speedup 1.0000x reference)
import functools

import jax
import jax.numpy as jnp
from jax import lax
from jax.experimental import pallas as pl
from jax.experimental.pallas import tpu as pltpu
from jax.experimental.pallas import tpu_sc as plsc

L = 16
NC = 2
NS = 16
NW = NC * NS
TOTAL = 16384 * 200
PER_W = TOTAL // NW
CHUNK = 25600
NCHUNK = PER_W // CHUNK

_mesh = plsc.VectorSubcoreMesh(core_axis_name="c", subcore_axis_name="s")


@functools.partial(
    pl.kernel,
    mesh=_mesh,
    out_type=jax.ShapeDtypeStruct((NW, L), jnp.float32),
    compiler_params=pltpu.CompilerParams(needs_layout_passes=False),
    scratch_types=[
        pltpu.VMEM((32,), jnp.float32),
        pltpu.VMEM((CHUNK,), jnp.float32),
        pltpu.VMEM((CHUNK,), jnp.int32),
        pltpu.VMEM((L,), jnp.float32),
    ],
)
def _wae_sc(x_hbm, t_hbm, table_hbm, out_hbm, table_v, xb, tb, pv):
    wid = lax.axis_index("s") * NC + lax.axis_index("c")
    base = wid * PER_W

    pltpu.sync_copy(table_hbm, table_v)

    acc0 = jnp.zeros((L,), jnp.float32)

    def chunk_body(c, acc):
        pltpu.sync_copy(x_hbm.at[pl.ds(base + c * CHUNK, CHUNK)], xb)
        pltpu.sync_copy(t_hbm.at[pl.ds(base + c * CHUNK, CHUNK)], tb)

        def body(i, a):
            s = pl.ds(i * L, L)
            xv = xb[s]
            tv = tb[s]
            w = plsc.load_gather(table_v, [tv])
            d = jnp.abs(xv - tv.astype(jnp.float32))
            return a + w * d

        return lax.fori_loop(0, CHUNK // L, body, acc)

    acc = lax.fori_loop(0, NCHUNK, chunk_body, acc0)
    pv[...] = acc
    pltpu.sync_copy(pv, out_hbm.at[wid])


def kernel(inputs, targets, class_weights):
    m = jnp.sum(class_weights > 0).astype(jnp.float32)
    C = jnp.where(m > 0, 1.0 / m, 1.0)
    table = jnp.pad(class_weights * C, (0, 32 - class_weights.shape[0]))
    x = inputs.reshape(-1)
    t = targets.astype(jnp.int32).reshape(-1)
    partials = _wae_sc(x, t, table)
    return jnp.sum(partials)


# speedup vs baseline: 1.2458x; 1.2458x over previous
import functools

import jax
import jax.numpy as jnp
from jax import lax
from jax.experimental import pallas as pl
from jax.experimental.pallas import tpu as pltpu
from jax.experimental.pallas import tpu_sc as plsc

L = 16
NC = 2
NS = 16
NW = NC * NS
TOTAL = 16384 * 200
PER_W = TOTAL // NW
CHUNK = 25600
NCHUNK = PER_W // CHUNK
NACC = 4

_mesh = plsc.VectorSubcoreMesh(core_axis_name="c", subcore_axis_name="s")


@functools.partial(
    pl.kernel,
    mesh=_mesh,
    out_type=jax.ShapeDtypeStruct((NW, L), jnp.float32),
    compiler_params=pltpu.CompilerParams(needs_layout_passes=False),
    scratch_types=[
        pltpu.VMEM((32,), jnp.float32),
        pltpu.VMEM((2, CHUNK), jnp.float32),
        pltpu.VMEM((2, CHUNK), jnp.int32),
        pltpu.VMEM((L,), jnp.float32),
        pltpu.SemaphoreType.DMA,
        pltpu.SemaphoreType.DMA,
    ],
)
def _wae_sc(x_hbm, t_hbm, table_hbm, out_hbm, table_v, xb, tb, pv, sem0, sem1):
    wid = lax.axis_index("s") * NC + lax.axis_index("c")
    base = wid * PER_W
    sems = (sem0, sem1)

    pltpu.sync_copy(table_hbm, table_v)

    def start(c):
        b = c % 2
        src = pl.ds(base + c * CHUNK, CHUNK)
        return (
            pltpu.async_copy(x_hbm.at[src], xb.at[b], sems[b]),
            pltpu.async_copy(t_hbm.at[src], tb.at[b], sems[b]),
        )

    inflight = start(0)
    accs = (jnp.zeros((L,), jnp.float32),) * NACC
    for c in range(NCHUNK):
        for h in inflight:
            h.wait()
        if c + 1 < NCHUNK:
            inflight = start(c + 1)
        b = c % 2

        def body(i, a):
            out = []
            for j in range(NACC):
                s = pl.ds(i + j * L, L)
                xv = xb[b, s]
                tv = tb[b, s]
                w = plsc.load_gather(table_v, [tv])
                out.append(a[j] + w * jnp.abs(xv - tv.astype(jnp.float32)))
            return tuple(out)

        accs = plsc.parallel_loop(0, CHUNK, NACC * L, unroll=4, carry=accs)(body)

    pv[...] = accs[0] + accs[1] + accs[2] + accs[3]
    pltpu.sync_copy(pv, out_hbm.at[wid])


def kernel(inputs, targets, class_weights):
    m = jnp.sum(class_weights > 0).astype(jnp.float32)
    C = jnp.where(m > 0, 1.0 / m, 1.0)
    table = jnp.pad(class_weights * C, (0, 32 - class_weights.shape[0]))
    x = inputs.reshape(-1)
    t = targets.astype(jnp.int32).reshape(-1)
    partials = _wae_sc(x, t, table)
    return jnp.sum(partials)


# speedup vs baseline: 1.7068x; 1.3701x over previous
import functools

import jax
import jax.numpy as jnp
from jax import lax
from jax.experimental import pallas as pl
from jax.experimental.pallas import tpu as pltpu
from jax.experimental.pallas import tpu_sc as plsc

L = 16
NC = 2
NS = 16
NW = NC * NS
NROW = 16384
NCOL = 200
ROWS_W = NROW // NW
RCHUNK = 32
NCHUNK = ROWS_W // RCHUNK
NFULL = NCOL // L
TAIL = NCOL - L
NACC = 4

_mesh = plsc.VectorSubcoreMesh(core_axis_name="c", subcore_axis_name="s")


@functools.partial(
    pl.kernel,
    mesh=_mesh,
    out_type=jax.ShapeDtypeStruct((NW, L), jnp.float32),
    compiler_params=pltpu.CompilerParams(needs_layout_passes=False),
    scratch_types=[
        pltpu.VMEM((32,), jnp.float32),
        pltpu.VMEM((2, RCHUNK, NCOL), jnp.float32),
        pltpu.VMEM((2, RCHUNK, NCOL), jnp.int32),
        pltpu.VMEM((L,), jnp.float32),
        pltpu.SemaphoreType.DMA,
        pltpu.SemaphoreType.DMA,
    ],
)
def _wae_sc(x_hbm, t_hbm, table_hbm, out_hbm, table_v, xb, tb, pv, sem0, sem1):
    wid = lax.axis_index("s") * NC + lax.axis_index("c")
    base = wid * ROWS_W
    sems = (sem0, sem1)

    pltpu.sync_copy(table_hbm, table_v)
    tail_keep = lax.iota(jnp.int32, L) >= (L - (NCOL - NFULL * L))

    def start(c):
        b = c % 2
        src = pl.ds(base + c * RCHUNK, RCHUNK)
        return (
            pltpu.async_copy(x_hbm.at[src], xb.at[b], sems[b]),
            pltpu.async_copy(t_hbm.at[src], tb.at[b], sems[b]),
        )

    inflight = start(0)
    accs = (jnp.zeros((L,), jnp.float32),) * NACC
    for c in range(NCHUNK):
        for h in inflight:
            h.wait()
        if c + 1 < NCHUNK:
            inflight = start(c + 1)
        b = c % 2

        def body(r, a):
            a = list(a)
            for j in range(NFULL + 1):
                s = pl.ds(j * L if j < NFULL else TAIL, L)
                xv = xb[b, r, s]
                tv = tb[b, r, s]
                w = plsc.load_gather(table_v, [tv])
                wd = w * jnp.abs(xv - tv.astype(jnp.float32))
                if j == NFULL:
                    wd = jnp.where(tail_keep, wd, 0.0)
                a[j % NACC] = a[j % NACC] + wd
            return tuple(a)

        accs = plsc.parallel_loop(0, RCHUNK, 1, unroll=2, carry=accs)(body)

    pv[...] = accs[0] + accs[1] + accs[2] + accs[3]
    pltpu.sync_copy(pv, out_hbm.at[wid])


def kernel(inputs, targets, class_weights):
    m = jnp.sum(class_weights > 0).astype(jnp.float32)
    C = jnp.where(m > 0, 1.0 / m, 1.0)
    table = jnp.pad(class_weights * C, (0, 32 - class_weights.shape[0]))
    partials = _wae_sc(inputs, targets.astype(jnp.int32), table)
    return jnp.sum(partials)
